# baseline (speedup 1.0000x reference)
import functools

import jax
import jax.numpy as jnp
from jax import lax
from jax.experimental import pallas as pl
from jax.experimental.pallas import tpu as pltpu
from jax.experimental.pallas import tpu_sc as plsc

N_DOM = 26
N_CONC = 100000
P_DIM = 16
BATCH_B = 16384

NW = 32
ROWS_PER_GATHER = 128
G_PER_CHUNK = 8
CHUNK = G_PER_CHUNK * ROWS_PER_GATHER
TOTAL_ROWS = BATCH_B * N_DOM
ROWS_PER_W = TOTAL_ROWS // NW
CHUNKS_PER_W = ROWS_PER_W // CHUNK


def _sc_gather(labels2d, table):
    mesh = plsc.VectorSubcoreMesh(core_axis_name="c", subcore_axis_name="s")

    @functools.partial(
        pl.kernel,
        mesh=mesh,
        out_type=jax.ShapeDtypeStruct((TOTAL_ROWS, P_DIM), jnp.float32),
        scratch_types=[
            pltpu.VMEM((G_PER_CHUNK, ROWS_PER_GATHER), jnp.int32),
            pltpu.VMEM((CHUNK, P_DIM), jnp.float32),
            pltpu.SemaphoreType.DMA,
        ],
        compiler_params=pltpu.CompilerParams(use_tc_tiling_on_sc=False),
    )
    def k(labels_hbm, table_hbm, out_hbm, idx_v, rows_v, sem):
        wid = lax.axis_index("s") * 2 + lax.axis_index("c")
        lane = lax.iota(jnp.int32, 16)

        def chunk_body(c, carry):
            base = pl.multiple_of(wid * ROWS_PER_W + c * CHUNK, CHUNK)
            lab_row0 = pl.multiple_of(base // ROWS_PER_GATHER, G_PER_CHUNK)
            pltpu.sync_copy(labels_hbm.at[pl.ds(lab_row0, G_PER_CHUNK)], idx_v)
            for j in range(G_PER_CHUNK):
                for kk in range(ROWS_PER_GATHER // 16):
                    pos = lane + (base + j * ROWS_PER_GATHER + kk * 16)
                    off = lax.rem(pos, N_DOM) * N_CONC
                    sl = (j, pl.ds(kk * 16, 16))
                    idx_v[sl] = idx_v[sl] + off
            copies = [
                pltpu.async_copy(
                    table_hbm.at[idx_v.at[j]],
                    rows_v.at[pl.ds(j * ROWS_PER_GATHER, ROWS_PER_GATHER)],
                    sem)
                for j in range(G_PER_CHUNK)
            ]
            for cp in copies:
                cp.wait()
            pltpu.sync_copy(rows_v, out_hbm.at[pl.ds(base, CHUNK)])
            return carry

        lax.fori_loop(0, CHUNKS_PER_W, chunk_body, 0)

    return k(labels2d, table)


def kernel(labels, pqc_params):
    labels2d = labels.astype(jnp.int32).reshape(-1, ROWS_PER_GATHER)
    table = pqc_params.reshape(N_DOM * N_CONC, P_DIM)
    out = _sc_gather(labels2d, table)
    return out.reshape(BATCH_B, N_DOM, P_DIM)
